# baseline (speedup 1.0000x reference)
import numpy as np
import jax
import jax.numpy as jnp
from jax import lax
from jax.experimental import pallas as pl
from jax.experimental.pallas import tpu as pltpu
from jax.experimental.pallas import tpu_sc as plsc

_B, _C, _H, _W = 4, 192, 224, 224
_HW = _H * _W

_SWAPPED = [
    2, 3, 4, 5, 6, 7, 8, 10, 11, 15, 16, 18, 19, 20, 22, 24, 29, 30, 31, 32,
    34, 35, 37, 39, 42, 43, 44, 45, 49, 50, 53, 54, 56, 58, 61, 63, 65, 67,
    69, 70, 72, 77, 78, 80, 81, 82, 83, 85, 90, 92, 94, 96, 99, 101, 102,
    108, 110, 111, 112, 114, 117, 118, 121, 123, 129, 130, 137, 138, 139,
    140, 142, 144, 147, 148, 152, 153, 155, 156, 157, 159, 163, 167, 169,
    173, 174, 175, 176, 177, 178, 179, 183, 184, 185, 186, 188, 189,
]
_MASK = np.zeros(_C, dtype=bool)
_MASK[np.asarray(_SWAPPED)] = True

_Q = 8
_CHUNK = _HW // _Q
_NROW = _B * _C * _Q
_NW = 32
_K = 8
_PER_W = _NROW // _NW
_NBLK = _PER_W // _K


def _build_tables():
    b, c, q = np.meshgrid(
        np.arange(_B), np.arange(_C), np.arange(_Q), indexing="ij"
    )
    b, c, q = b.ravel(), c.ravel(), q.ravel()
    src = (b * _C + c) * _Q + q
    swapped = _MASK[c]
    dstA = np.where(swapped, (b * 2 * _C + _C + c), (b * 2 * _C + c)) * _Q + q
    dstB = np.where(swapped, (b * 2 * _C + c), (b * 2 * _C + _C + c)) * _Q + q
    shape = (_NW, _NBLK, _K)
    return (
        src.reshape(shape).astype(np.int32),
        dstA.reshape(shape).astype(np.int32),
        src.reshape(shape).astype(np.int32),
        dstB.reshape(shape).astype(np.int32),
    )


_SRC_A, _DST_A, _SRC_B, _DST_B = _build_tables()


def _sc_body(
    f1r, f2r, sA, dA, sB, dB, out,
    idx_s0, idx_d0, idx_s1, idx_d1, buf0, buf1, sem0, sem1,
):
    w = lax.axis_index("s") * 2 + lax.axis_index("c")

    for src_hbm, S, D in ((f1r, sA, dA), (f2r, sB, dB)):
        idx = ((idx_s0, idx_d0, buf0, sem0), (idx_s1, idx_d1, buf1, sem1))

        def start(j, par):
            idx_s, idx_d, buf, sem = idx[par]
            pltpu.sync_copy(S.at[w, j], idx_s)
            pltpu.sync_copy(D.at[w, j], idx_d)
            return pltpu.async_copy(src_hbm.at[idx_s], buf, sem)

        def drain(j, par):
            idx_s, idx_d, buf, sem = idx[par]
            pltpu.async_copy(buf, out.at[idx_d], sem).wait()

        cp = start(0, 0)
        for j in range(_NBLK):
            cp.wait()
            if j + 1 < _NBLK:
                nxt = start(j + 1, (j + 1) % 2)
            drain(j, j % 2)
            if j + 1 < _NBLK:
                cp = nxt


def kernel(f1, f2):
    B, C, H, W = f1.shape
    a = f1.reshape(_NROW, _CHUNK)
    b = f2.reshape(_NROW, _CHUNK)

    mesh = plsc.VectorSubcoreMesh(core_axis_name="c", subcore_axis_name="s")
    run = pl.kernel(
        _sc_body,
        mesh=mesh,
        out_type=jax.ShapeDtypeStruct((2 * _NROW, _CHUNK), f1.dtype),
        scratch_types=[
            pltpu.VMEM((_K,), jnp.int32),
            pltpu.VMEM((_K,), jnp.int32),
            pltpu.VMEM((_K,), jnp.int32),
            pltpu.VMEM((_K,), jnp.int32),
            pltpu.VMEM((_K, _CHUNK), jnp.float32),
            pltpu.VMEM((_K, _CHUNK), jnp.float32),
            pltpu.SemaphoreType.DMA,
            pltpu.SemaphoreType.DMA,
        ],
    )
    out = run(
        a, b,
        jnp.asarray(_SRC_A), jnp.asarray(_DST_A),
        jnp.asarray(_SRC_B), jnp.asarray(_DST_B),
    )
    return out.reshape(B, 2 * C, H, W)


# speedup vs baseline: 1.0287x; 1.0287x over previous
import numpy as np
import jax
import jax.numpy as jnp
from jax import lax
from jax.experimental import pallas as pl
from jax.experimental.pallas import tpu as pltpu
from jax.experimental.pallas import tpu_sc as plsc

_B, _C, _H, _W = 4, 192, 224, 224
_HW = _H * _W

_SWAPPED = [
    2, 3, 4, 5, 6, 7, 8, 10, 11, 15, 16, 18, 19, 20, 22, 24, 29, 30, 31, 32,
    34, 35, 37, 39, 42, 43, 44, 45, 49, 50, 53, 54, 56, 58, 61, 63, 65, 67,
    69, 70, 72, 77, 78, 80, 81, 82, 83, 85, 90, 92, 94, 96, 99, 101, 102,
    108, 110, 111, 112, 114, 117, 118, 121, 123, 129, 130, 137, 138, 139,
    140, 142, 144, 147, 148, 152, 153, 155, 156, 157, 159, 163, 167, 169,
    173, 174, 175, 176, 177, 178, 179, 183, 184, 185, 186, 188, 189,
]
_MASK = np.zeros(_C, dtype=bool)
_MASK[np.asarray(_SWAPPED)] = True

_Q = 8
_CHUNK = _HW // _Q
_NROW = _B * _C * _Q
_NW = 32
_K = 8
_PER_W = _NROW // _NW
_NBLK = _PER_W // _K


def _build_tables():
    b, c, q = np.meshgrid(
        np.arange(_B), np.arange(_C), np.arange(_Q), indexing="ij"
    )
    b, c, q = b.ravel(), c.ravel(), q.ravel()
    src = (b * _C + c) * _Q + q
    swapped = _MASK[c]
    dstA = np.where(swapped, (b * 2 * _C + _C + c), (b * 2 * _C + c)) * _Q + q
    dstB = np.where(swapped, (b * 2 * _C + c), (b * 2 * _C + _C + c)) * _Q + q
    shape = (_NW, _NBLK, _K)
    return (
        src.reshape(shape).astype(np.int32),
        dstA.reshape(shape).astype(np.int32),
        src.reshape(shape).astype(np.int32),
        dstB.reshape(shape).astype(np.int32),
    )


_SRC_A, _DST_A, _SRC_B, _DST_B = _build_tables()


def _sc_body(
    f1r, f2r, sA, dA, sB, dB, out,
    idx_sv, idx_dv, buf0, buf1, sem0, sem1,
):
    w = lax.axis_index("s") * 2 + lax.axis_index("c")

    for src_hbm, S, D in ((f1r, sA, dA), (f2r, sB, dB)):
        pltpu.sync_copy(S.at[w], idx_sv)
        pltpu.sync_copy(D.at[w], idx_dv)
        bufs = ((buf0, sem0), (buf1, sem1))

        def start(j, par):
            buf, sem = bufs[par]
            return pltpu.async_copy(src_hbm.at[idx_sv.at[j]], buf, sem)

        def drain(j, par):
            buf, sem = bufs[par]
            pltpu.async_copy(buf, out.at[idx_dv.at[j]], sem).wait()

        cp = start(0, 0)
        for j in range(_NBLK):
            cp.wait()
            if j + 1 < _NBLK:
                nxt = start(j + 1, (j + 1) % 2)
            drain(j, j % 2)
            if j + 1 < _NBLK:
                cp = nxt


def kernel(f1, f2):
    B, C, H, W = f1.shape
    a = f1.reshape(_NROW, _CHUNK)
    b = f2.reshape(_NROW, _CHUNK)

    mesh = plsc.VectorSubcoreMesh(core_axis_name="c", subcore_axis_name="s")
    run = pl.kernel(
        _sc_body,
        mesh=mesh,
        out_type=jax.ShapeDtypeStruct((2 * _NROW, _CHUNK), f1.dtype),
        scratch_types=[
            pltpu.VMEM((_NBLK, _K), jnp.int32),
            pltpu.VMEM((_NBLK, _K), jnp.int32),
            pltpu.VMEM((_K, _CHUNK), jnp.float32),
            pltpu.VMEM((_K, _CHUNK), jnp.float32),
            pltpu.SemaphoreType.DMA,
            pltpu.SemaphoreType.DMA,
        ],
    )
    out = run(
        a, b,
        jnp.asarray(_SRC_A), jnp.asarray(_DST_A),
        jnp.asarray(_SRC_B), jnp.asarray(_DST_B),
    )
    return out.reshape(B, 2 * C, H, W)


# speedup vs baseline: 2.6983x; 2.6231x over previous
import numpy as np
import jax
import jax.numpy as jnp
from jax.experimental import pallas as pl
from jax.experimental.pallas import tpu as pltpu

_C = 192

_SWAPPED = [
    2, 3, 4, 5, 6, 7, 8, 10, 11, 15, 16, 18, 19, 20, 22, 24, 29, 30, 31, 32,
    34, 35, 37, 39, 42, 43, 44, 45, 49, 50, 53, 54, 56, 58, 61, 63, 65, 67,
    69, 70, 72, 77, 78, 80, 81, 82, 83, 85, 90, 92, 94, 96, 99, 101, 102,
    108, 110, 111, 112, 114, 117, 118, 121, 123, 129, 130, 137, 138, 139,
    140, 142, 144, 147, 148, 152, 153, 155, 156, 157, 159, 163, 167, 169,
    173, 174, 175, 176, 177, 178, 179, 183, 184, 185, 186, 188, 189,
]
_MASK = np.zeros(_C, dtype=bool)
_MASK[np.asarray(_SWAPPED)] = True

_CBLK = 4


def _body(mask_ref, f1_ref, f2_ref, o_ref):
    i = pl.program_id(0)
    for j in range(_CBLK):
        swapped = mask_ref[i * _CBLK + j] != 0

        @pl.when(swapped)
        def _():
            o_ref[:, 0, j] = f2_ref[:, j]
            o_ref[:, 1, j] = f1_ref[:, j]

        @pl.when(jnp.logical_not(swapped))
        def _():
            o_ref[:, 0, j] = f1_ref[:, j]
            o_ref[:, 1, j] = f2_ref[:, j]


@jax.jit
def kernel(f1, f2):
    B, C, H, W = f1.shape

    grid_spec = pltpu.PrefetchScalarGridSpec(
        num_scalar_prefetch=1,
        grid=(C // _CBLK,),
        in_specs=[
            pl.BlockSpec((B, _CBLK, H, W), lambda i, m: (0, i, 0, 0)),
            pl.BlockSpec((B, _CBLK, H, W), lambda i, m: (0, i, 0, 0)),
        ],
        out_specs=pl.BlockSpec(
            (B, 2, _CBLK, H, W), lambda i, m: (0, 0, i, 0, 0)
        ),
    )
    out = pl.pallas_call(
        _body,
        grid_spec=grid_spec,
        out_shape=jax.ShapeDtypeStruct((B, 2, C, H, W), f1.dtype),
        compiler_params=pltpu.CompilerParams(
            dimension_semantics=("arbitrary",),
        ),
    )(jnp.asarray(_MASK, jnp.int32), f1, f2)
    return out.reshape(B, 2 * C, H, W)


# speedup vs baseline: 2.7041x; 1.0022x over previous
import numpy as np
import jax
import jax.numpy as jnp
from jax.experimental import pallas as pl
from jax.experimental.pallas import tpu as pltpu

_C = 192

_SWAPPED = [
    2, 3, 4, 5, 6, 7, 8, 10, 11, 15, 16, 18, 19, 20, 22, 24, 29, 30, 31, 32,
    34, 35, 37, 39, 42, 43, 44, 45, 49, 50, 53, 54, 56, 58, 61, 63, 65, 67,
    69, 70, 72, 77, 78, 80, 81, 82, 83, 85, 90, 92, 94, 96, 99, 101, 102,
    108, 110, 111, 112, 114, 117, 118, 121, 123, 129, 130, 137, 138, 139,
    140, 142, 144, 147, 148, 152, 153, 155, 156, 157, 159, 163, 167, 169,
    173, 174, 175, 176, 177, 178, 179, 183, 184, 185, 186, 188, 189,
]
_MASK = np.zeros(_C, dtype=bool)
_MASK[np.asarray(_SWAPPED)] = True

_CBLK = 8


def _body(mask_ref, f1_ref, f2_ref, o_ref):
    i = pl.program_id(0)
    for j in range(_CBLK):
        swapped = mask_ref[i * _CBLK + j] != 0

        @pl.when(swapped)
        def _():
            o_ref[:, 0, j] = f2_ref[:, j]
            o_ref[:, 1, j] = f1_ref[:, j]

        @pl.when(jnp.logical_not(swapped))
        def _():
            o_ref[:, 0, j] = f1_ref[:, j]
            o_ref[:, 1, j] = f2_ref[:, j]


@jax.jit
def kernel(f1, f2):
    B, C, H, W = f1.shape

    grid_spec = pltpu.PrefetchScalarGridSpec(
        num_scalar_prefetch=1,
        grid=(C // _CBLK,),
        in_specs=[
            pl.BlockSpec((B, _CBLK, H, W), lambda i, m: (0, i, 0, 0)),
            pl.BlockSpec((B, _CBLK, H, W), lambda i, m: (0, i, 0, 0)),
        ],
        out_specs=pl.BlockSpec(
            (B, 2, _CBLK, H, W), lambda i, m: (0, 0, i, 0, 0)
        ),
    )
    out = pl.pallas_call(
        _body,
        grid_spec=grid_spec,
        out_shape=jax.ShapeDtypeStruct((B, 2, C, H, W), f1.dtype),
        compiler_params=pltpu.CompilerParams(
            dimension_semantics=("arbitrary",),
        ),
    )(jnp.asarray(_MASK, jnp.int32), f1, f2)
    return out.reshape(B, 2 * C, H, W)
